# Initial kernel scaffold; baseline (speedup 1.0000x reference)
import functools

import jax
import jax.numpy as jnp
from jax import lax
from jax.experimental import pallas as pl
from jax.experimental.pallas import tpu as pltpu
from jax.experimental.pallas import tpu_sc as plsc

_NC = 2
_NS = 16
_NW = _NC * _NS


def _sc_edge_scatter(table, gidx, sidx, zeros, n_rows_out, n_edges, batch):
    d = table.shape[1]
    e_per_w = n_edges // _NW
    n_iter = e_per_w // batch
    rows_per_tile = n_rows_out // _NS

    mesh = plsc.VectorSubcoreMesh(core_axis_name="c", subcore_axis_name="s")

    @functools.partial(
        pl.kernel,
        out_type=jax.ShapeDtypeStruct((_NC, n_rows_out, d), jnp.float32),
        mesh=mesh,
        scratch_types=[
            pltpu.VMEM((batch,), jnp.int32),
            pltpu.VMEM((batch,), jnp.int32),
            pltpu.VMEM((batch, d), jnp.float32),
            pltpu.VMEM_SHARED((n_rows_out, d), jnp.float32),
            pltpu.SemaphoreType.DMA,
        ],
    )
    def k(table_hbm, gidx_hbm, sidx_hbm, zeros_hbm, out_hbm, gi_v, si_v, rows_v, acc_sh, sem):
        c = lax.axis_index("c")
        s = lax.axis_index("s")
        wid = s * _NC + c

        r0 = s * rows_per_tile
        pltpu.sync_copy(zeros_hbm.at[pl.ds(r0, rows_per_tile)],
                        acc_sh.at[pl.ds(r0, rows_per_tile)])
        plsc.subcore_barrier()

        def body(j, carry):
            base = pl.multiple_of(wid * e_per_w + j * batch, batch)
            pltpu.sync_copy(gidx_hbm.at[pl.ds(base, batch)], gi_v)
            pltpu.sync_copy(sidx_hbm.at[pl.ds(base, batch)], si_v)
            pltpu.async_copy(table_hbm.at[gi_v], rows_v, sem).wait()
            pltpu.sync_copy(rows_v, acc_sh.at[si_v], add=True)
            return carry

        lax.fori_loop(0, n_iter, body, 0)
        plsc.subcore_barrier()

        pltpu.sync_copy(acc_sh.at[pl.ds(r0, rows_per_tile)],
                        out_hbm.at[c, pl.ds(r0, rows_per_tile)])

    return k(table, gidx, sidx, zeros)


def _tc_linear(x, w, b, block_rows):
    r, d = x.shape

    def body(x_ref, w_ref, b_ref, o_ref):
        o_ref[...] = (
            jnp.dot(x_ref[...], w_ref[...].T, preferred_element_type=jnp.float32)
            + b_ref[...]
        )

    return pl.pallas_call(
        body,
        grid=(r // block_rows,),
        in_specs=[
            pl.BlockSpec((block_rows, d), lambda i: (i, 0)),
            pl.BlockSpec((d, d), lambda i: (0, 0)),
            pl.BlockSpec((1, d), lambda i: (0, 0)),
        ],
        out_specs=pl.BlockSpec((block_rows, d), lambda i: (i, 0)),
        out_shape=jax.ShapeDtypeStruct((r, d), jnp.float32),
    )(x, w, b.reshape(1, d))


def _tc_fuse(acc, x_m, agg, w_l, b_l, w_r, w_out, b_out, bn_weight, bn_bias, block_rows):
    m, d = x_m.shape
    inv = float(1.0 / (1.00001 ** 0.5) if False else (1.0 + 1e-5) ** -0.5)

    def body(acc_ref, xm_ref, ag_ref, wl_ref, bl_ref, wr_ref, wo_ref, bo_ref,
             bnw_ref, bnb_ref, o_ref):
        hs = acc_ref[0] + acc_ref[1]
        xm = xm_ref[...]
        hseq = (
            jnp.dot(ag_ref[...], wl_ref[...].T, preferred_element_type=jnp.float32)
            + bl_ref[...]
            + jnp.dot(xm, wr_ref[...].T, preferred_element_type=jnp.float32)
        )
        wo = wo_ref[...]
        h = (
            jnp.dot(hs, wo[:, :d].T, preferred_element_type=jnp.float32)
            + jnp.dot(xm, wo[:, d:2 * d].T, preferred_element_type=jnp.float32)
            + jnp.dot(hseq, wo[:, 2 * d:].T, preferred_element_type=jnp.float32)
            + bo_ref[...]
        )
        o_ref[...] = h * (bnw_ref[...] * inv) + bnb_ref[...]

    d_out = w_out.shape[0]
    return pl.pallas_call(
        body,
        grid=(m // block_rows,),
        in_specs=[
            pl.BlockSpec((2, block_rows, d), lambda i: (0, i, 0)),
            pl.BlockSpec((block_rows, d), lambda i: (i, 0)),
            pl.BlockSpec((block_rows, d), lambda i: (i, 0)),
            pl.BlockSpec((d, d), lambda i: (0, 0)),
            pl.BlockSpec((1, d), lambda i: (0, 0)),
            pl.BlockSpec((d, d), lambda i: (0, 0)),
            pl.BlockSpec((d_out, 3 * d), lambda i: (0, 0)),
            pl.BlockSpec((1, d_out), lambda i: (0, 0)),
            pl.BlockSpec((1, d_out), lambda i: (0, 0)),
            pl.BlockSpec((1, d_out), lambda i: (0, 0)),
        ],
        out_specs=pl.BlockSpec((block_rows, d_out), lambda i: (i, 0)),
        out_shape=jax.ShapeDtypeStruct((m, d_out), jnp.float32),
    )(acc, x_m, agg, w_l, b_l.reshape(1, d), w_r, w_out, b_out.reshape(1, d_out),
      bn_weight.reshape(1, d_out), bn_bias.reshape(1, d_out))


def _tc_sum_partials(p, block_rows):
    _, r, d = p.shape

    def body(p_ref, o_ref):
        o_ref[...] = p_ref[0] + p_ref[1]

    return pl.pallas_call(
        body,
        grid=(r // block_rows,),
        in_specs=[pl.BlockSpec((2, block_rows, d), lambda i: (0, i, 0))],
        out_specs=pl.BlockSpec((block_rows, d), lambda i: (i, 0)),
        out_shape=jax.ShapeDtypeStruct((r, d), jnp.float32),
    )(p)


def kernel(x_metrical, x, edge_index, batch, W_neigh, b_neigh, W_l, b_l, W_r,
           W_out, b_out, bn_weight, bn_bias):
    m, d = x_metrical.shape
    n = x.shape[0]
    e = edge_index.shape[1]

    src = edge_index[0].astype(jnp.int32)
    dst = edge_index[1].astype(jnp.int32)

    h_neigh = _tc_linear(x, W_neigh, b_neigh, block_rows=1000)

    zeros_m = jnp.zeros((m, d), jnp.float32)
    acc = _sc_edge_scatter(h_neigh, src, dst, zeros_m, m, e, batch=80)

    agg = jnp.concatenate([jnp.zeros((1, d), jnp.float32), x_metrical[:-1]], axis=0)
    h = _tc_fuse(acc, x_metrical, agg, W_l, b_l, W_r, W_out, b_out,
                 bn_weight, bn_bias, block_rows=1000)

    zeros_n = jnp.zeros((n, d), jnp.float32)
    part = _sc_edge_scatter(h, dst, src, zeros_n, n, e, batch=80)

    return _tc_sum_partials(part, block_rows=1000)


# speedup vs baseline: 4.6130x; 4.6130x over previous
import functools

import jax
import jax.numpy as jnp
from jax import lax
from jax.experimental import pallas as pl
from jax.experimental.pallas import tpu as pltpu
from jax.experimental.pallas import tpu_sc as plsc

_NC = 2
_NS = 16
_NW = _NC * _NS


def _sc_edge_scatter(table, gidx, sidx, n_rows_out, n_edges, batch):
    d = table.shape[1]
    e_per_w = n_edges // _NW
    n_iter = e_per_w // batch
    rows_per_tile = -(-n_rows_out // (_NS * 8)) * 8
    n_pad = rows_per_tile * _NS

    mesh = plsc.VectorSubcoreMesh(core_axis_name="c", subcore_axis_name="s")

    @functools.partial(
        pl.kernel,
        out_type=jax.ShapeDtypeStruct((_NC, n_pad, d), jnp.float32),
        mesh=mesh,
        scratch_types=[
            pltpu.VMEM((batch,), jnp.int32),
            pltpu.VMEM((batch,), jnp.int32),
            pltpu.VMEM((batch, d), jnp.float32),
            pltpu.VMEM_SHARED((n_pad, d), jnp.float32),
            pltpu.SemaphoreType.DMA,
        ],
    )
    def k(table_hbm, gidx_hbm, sidx_hbm, zeros_hbm, out_hbm, gi_v, si_v, rows_v, acc_sh, sem):
        c = lax.axis_index("c")
        s = lax.axis_index("s")
        wid = s * _NC + c

        r0 = s * rows_per_tile
        pltpu.sync_copy(zeros_hbm.at[pl.ds(r0, rows_per_tile)],
                        acc_sh.at[pl.ds(r0, rows_per_tile)])
        plsc.subcore_barrier()

        def body(j, carry):
            base = pl.multiple_of(wid * e_per_w + j * batch, batch)
            pltpu.sync_copy(gidx_hbm.at[pl.ds(base, batch)], gi_v)
            pltpu.sync_copy(sidx_hbm.at[pl.ds(base, batch)], si_v)
            pltpu.async_copy(table_hbm.at[gi_v], rows_v, sem).wait()
            pltpu.sync_copy(rows_v, acc_sh.at[si_v], add=True)
            return carry

        lax.fori_loop(0, n_iter, body, 0)
        plsc.subcore_barrier()

        pltpu.sync_copy(acc_sh.at[pl.ds(r0, rows_per_tile)],
                        out_hbm.at[c, pl.ds(r0, rows_per_tile)])

    return k(table, gidx, sidx, jnp.zeros((n_pad, d), jnp.float32))


def _tc_linear(x, w, b, block_rows):
    r, d = x.shape

    def body(x_ref, w_ref, b_ref, o_ref):
        o_ref[...] = (
            jnp.dot(x_ref[...], w_ref[...].T, preferred_element_type=jnp.float32)
            + b_ref[...]
        )

    return pl.pallas_call(
        body,
        grid=(r // block_rows,),
        in_specs=[
            pl.BlockSpec((block_rows, d), lambda i: (i, 0)),
            pl.BlockSpec((d, d), lambda i: (0, 0)),
            pl.BlockSpec((1, d), lambda i: (0, 0)),
        ],
        out_specs=pl.BlockSpec((block_rows, d), lambda i: (i, 0)),
        out_shape=jax.ShapeDtypeStruct((r, d), jnp.float32),
    )(x, w, b.reshape(1, d))


def _tc_fuse(acc, x_m, agg, w_l, b_l, w_r, w_out, b_out, bn_weight, bn_bias, block_rows):
    m, d = x_m.shape
    inv = (1.0 + 1e-5) ** -0.5

    def body(acc_ref, xm_ref, ag_ref, wl_ref, bl_ref, wr_ref, wo_ref, bo_ref,
             bnw_ref, bnb_ref, o_ref):
        hs = acc_ref[0] + acc_ref[1]
        xm = xm_ref[...]
        hseq = (
            jnp.dot(ag_ref[...], wl_ref[...].T, preferred_element_type=jnp.float32)
            + bl_ref[...]
            + jnp.dot(xm, wr_ref[...].T, preferred_element_type=jnp.float32)
        )
        wo = wo_ref[...]
        h = (
            jnp.dot(hs, wo[:, :d].T, preferred_element_type=jnp.float32)
            + jnp.dot(xm, wo[:, d:2 * d].T, preferred_element_type=jnp.float32)
            + jnp.dot(hseq, wo[:, 2 * d:].T, preferred_element_type=jnp.float32)
            + bo_ref[...]
        )
        o_ref[...] = h * (bnw_ref[...] * inv) + bnb_ref[...]

    d_out = w_out.shape[0]
    return pl.pallas_call(
        body,
        grid=(m // block_rows,),
        in_specs=[
            pl.BlockSpec((2, block_rows, d), lambda i: (0, i, 0)),
            pl.BlockSpec((block_rows, d), lambda i: (i, 0)),
            pl.BlockSpec((block_rows, d), lambda i: (i, 0)),
            pl.BlockSpec((d, d), lambda i: (0, 0)),
            pl.BlockSpec((1, d), lambda i: (0, 0)),
            pl.BlockSpec((d, d), lambda i: (0, 0)),
            pl.BlockSpec((d_out, 3 * d), lambda i: (0, 0)),
            pl.BlockSpec((1, d_out), lambda i: (0, 0)),
            pl.BlockSpec((1, d_out), lambda i: (0, 0)),
            pl.BlockSpec((1, d_out), lambda i: (0, 0)),
        ],
        out_specs=pl.BlockSpec((block_rows, d_out), lambda i: (i, 0)),
        out_shape=jax.ShapeDtypeStruct((m, d_out), jnp.float32),
    )(acc, x_m, agg, w_l, b_l.reshape(1, d), w_r, w_out, b_out.reshape(1, d_out),
      bn_weight.reshape(1, d_out), bn_bias.reshape(1, d_out))


def _tc_sum_partials(p, r, block_rows):
    d = p.shape[2]

    def body(p_ref, o_ref):
        o_ref[...] = p_ref[0] + p_ref[1]

    return pl.pallas_call(
        body,
        grid=(r // block_rows,),
        in_specs=[pl.BlockSpec((2, block_rows, d), lambda i: (0, i, 0))],
        out_specs=pl.BlockSpec((block_rows, d), lambda i: (i, 0)),
        out_shape=jax.ShapeDtypeStruct((r, d), jnp.float32),
    )(p)


def kernel(x_metrical, x, edge_index, batch, W_neigh, b_neigh, W_l, b_l, W_r,
           W_out, b_out, bn_weight, bn_bias):
    m, d = x_metrical.shape
    n = x.shape[0]
    e = edge_index.shape[1]

    src = edge_index[0].astype(jnp.int32)
    dst = edge_index[1].astype(jnp.int32)

    h_neigh = _tc_linear(x, W_neigh, b_neigh, block_rows=1000)

    acc = _sc_edge_scatter(h_neigh, src, dst, m, e, batch=80)

    agg = jnp.concatenate([jnp.zeros((1, d), jnp.float32), x_metrical[:-1]], axis=0)
    h = _tc_fuse(acc, x_metrical, agg, W_l, b_l, W_r, W_out, b_out,
                 bn_weight, bn_bias, block_rows=1000)

    part = _sc_edge_scatter(h, dst, src, n, e, batch=80)

    return _tc_sum_partials(part, n, block_rows=1000)


# speedup vs baseline: 9.8168x; 2.1281x over previous
import functools

import jax
import jax.numpy as jnp
from jax import lax
from jax.experimental import pallas as pl
from jax.experimental.pallas import tpu as pltpu
from jax.experimental.pallas import tpu_sc as plsc

_NC = 2
_NS = 16
_NW = _NC * _NS


def _sc_edge_scatter(table, gidx, sidx, n_rows_out, n_edges, batch):
    d = table.shape[1]
    e_per_w = n_edges // _NW
    n_iter = e_per_w // batch
    n_chunks = 5
    n_sub = n_iter // n_chunks
    rows_per_tile = -(-n_rows_out // (_NS * 8)) * 8
    n_pad = rows_per_tile * _NS

    mesh = plsc.VectorSubcoreMesh(core_axis_name="c", subcore_axis_name="s")

    @functools.partial(
        pl.kernel,
        out_type=jax.ShapeDtypeStruct((_NC, n_pad, d), jnp.float32),
        mesh=mesh,
        scratch_types=[
            pltpu.VMEM((n_sub, batch), jnp.int32),
            pltpu.VMEM((n_sub, batch), jnp.int32),
            pltpu.VMEM((batch, d), jnp.float32),
            pltpu.VMEM((batch, d), jnp.float32),
            pltpu.VMEM_SHARED((n_pad, d), jnp.float32),
            pltpu.SemaphoreType.DMA,
            pltpu.SemaphoreType.DMA,
        ],
    )
    def k(table_hbm, gidx_hbm, sidx_hbm, zeros_hbm, out_hbm,
          gi_v, si_v, rows0, rows1, acc_sh, sem0, sem1):
        c = lax.axis_index("c")
        s = lax.axis_index("s")
        wid = s * _NC + c

        r0 = s * rows_per_tile
        pltpu.sync_copy(zeros_hbm.at[pl.ds(r0, rows_per_tile)],
                        acc_sh.at[pl.ds(r0, rows_per_tile)])
        plsc.subcore_barrier()

        def chunk(u, carry):
            pltpu.sync_copy(gidx_hbm.at[wid, pl.ds(u * n_sub, n_sub)], gi_v)
            pltpu.sync_copy(sidx_hbm.at[wid, pl.ds(u * n_sub, n_sub)], si_v)
            pltpu.async_copy(table_hbm.at[gi_v.at[0]], rows0, sem0)
            pltpu.async_copy(table_hbm.at[gi_v.at[1]], rows1, sem1)

            def body(k2, c2):
                j0 = k2 * 2
                pltpu.make_async_copy(table_hbm.at[gi_v.at[0]], rows0, sem0).wait()
                pltpu.sync_copy(rows0, acc_sh.at[si_v.at[j0]], add=True)
                jn0 = jnp.minimum(j0 + 2, n_sub - 1)
                pltpu.async_copy(table_hbm.at[gi_v.at[jn0]], rows0, sem0)
                pltpu.make_async_copy(table_hbm.at[gi_v.at[0]], rows1, sem1).wait()
                pltpu.sync_copy(rows1, acc_sh.at[si_v.at[j0 + 1]], add=True)
                jn1 = jnp.minimum(j0 + 3, n_sub - 1)
                pltpu.async_copy(table_hbm.at[gi_v.at[jn1]], rows1, sem1)
                return c2

            lax.fori_loop(0, n_sub // 2, body, 0)
            pltpu.make_async_copy(table_hbm.at[gi_v.at[0]], rows0, sem0).wait()
            if n_sub % 2 == 1:
                pltpu.sync_copy(rows0, acc_sh.at[si_v.at[n_sub - 1]], add=True)
            pltpu.make_async_copy(table_hbm.at[gi_v.at[0]], rows1, sem1).wait()
            return carry

        lax.fori_loop(0, n_chunks, chunk, 0)
        plsc.subcore_barrier()

        pltpu.sync_copy(acc_sh.at[pl.ds(r0, rows_per_tile)],
                        out_hbm.at[c, pl.ds(r0, rows_per_tile)])

    return k(table, gidx.reshape(_NW, n_iter, batch), sidx.reshape(_NW, n_iter, batch),
             jnp.zeros((n_pad, d), jnp.float32))


def _tc_linear(x, w, b, block_rows):
    r, d = x.shape

    def body(x_ref, w_ref, b_ref, o_ref):
        o_ref[...] = (
            jnp.dot(x_ref[...], w_ref[...].T, preferred_element_type=jnp.float32)
            + b_ref[...]
        )

    return pl.pallas_call(
        body,
        grid=(r // block_rows,),
        in_specs=[
            pl.BlockSpec((block_rows, d), lambda i: (i, 0)),
            pl.BlockSpec((d, d), lambda i: (0, 0)),
            pl.BlockSpec((1, d), lambda i: (0, 0)),
        ],
        out_specs=pl.BlockSpec((block_rows, d), lambda i: (i, 0)),
        out_shape=jax.ShapeDtypeStruct((r, d), jnp.float32),
    )(x, w, b.reshape(1, d))


def _tc_fuse(acc, x_m, agg, w_l, b_l, w_r, w_out, b_out, bn_weight, bn_bias, block_rows):
    m, d = x_m.shape
    inv = (1.0 + 1e-5) ** -0.5

    def body(acc_ref, xm_ref, ag_ref, wl_ref, bl_ref, wr_ref, wo_ref, bo_ref,
             bnw_ref, bnb_ref, o_ref):
        hs = acc_ref[0] + acc_ref[1]
        xm = xm_ref[...]
        hseq = (
            jnp.dot(ag_ref[...], wl_ref[...].T, preferred_element_type=jnp.float32)
            + bl_ref[...]
            + jnp.dot(xm, wr_ref[...].T, preferred_element_type=jnp.float32)
        )
        wo = wo_ref[...]
        h = (
            jnp.dot(hs, wo[:, :d].T, preferred_element_type=jnp.float32)
            + jnp.dot(xm, wo[:, d:2 * d].T, preferred_element_type=jnp.float32)
            + jnp.dot(hseq, wo[:, 2 * d:].T, preferred_element_type=jnp.float32)
            + bo_ref[...]
        )
        o_ref[...] = h * (bnw_ref[...] * inv) + bnb_ref[...]

    d_out = w_out.shape[0]
    return pl.pallas_call(
        body,
        grid=(m // block_rows,),
        in_specs=[
            pl.BlockSpec((2, block_rows, d), lambda i: (0, i, 0)),
            pl.BlockSpec((block_rows, d), lambda i: (i, 0)),
            pl.BlockSpec((block_rows, d), lambda i: (i, 0)),
            pl.BlockSpec((d, d), lambda i: (0, 0)),
            pl.BlockSpec((1, d), lambda i: (0, 0)),
            pl.BlockSpec((d, d), lambda i: (0, 0)),
            pl.BlockSpec((d_out, 3 * d), lambda i: (0, 0)),
            pl.BlockSpec((1, d_out), lambda i: (0, 0)),
            pl.BlockSpec((1, d_out), lambda i: (0, 0)),
            pl.BlockSpec((1, d_out), lambda i: (0, 0)),
        ],
        out_specs=pl.BlockSpec((block_rows, d_out), lambda i: (i, 0)),
        out_shape=jax.ShapeDtypeStruct((m, d_out), jnp.float32),
    )(acc, x_m, agg, w_l, b_l.reshape(1, d), w_r, w_out, b_out.reshape(1, d_out),
      bn_weight.reshape(1, d_out), bn_bias.reshape(1, d_out))


def _tc_sum_partials(p, r, block_rows):
    d = p.shape[2]

    def body(p_ref, o_ref):
        o_ref[...] = p_ref[0] + p_ref[1]

    return pl.pallas_call(
        body,
        grid=(r // block_rows,),
        in_specs=[pl.BlockSpec((2, block_rows, d), lambda i: (0, i, 0))],
        out_specs=pl.BlockSpec((block_rows, d), lambda i: (i, 0)),
        out_shape=jax.ShapeDtypeStruct((r, d), jnp.float32),
    )(p)


def kernel(x_metrical, x, edge_index, batch, W_neigh, b_neigh, W_l, b_l, W_r,
           W_out, b_out, bn_weight, bn_bias):
    m, d = x_metrical.shape
    n = x.shape[0]
    e = edge_index.shape[1]

    src = edge_index[0].astype(jnp.int32)
    dst = edge_index[1].astype(jnp.int32)

    h_neigh = _tc_linear(x, W_neigh, b_neigh, block_rows=1000)

    acc = _sc_edge_scatter(h_neigh, src, dst, m, e, batch=125)

    agg = jnp.concatenate([jnp.zeros((1, d), jnp.float32), x_metrical[:-1]], axis=0)
    h = _tc_fuse(acc, x_metrical, agg, W_l, b_l, W_r, W_out, b_out,
                 bn_weight, bn_bias, block_rows=1000)

    part = _sc_edge_scatter(h, dst, src, n, e, batch=125)

    return _tc_sum_partials(part, n, block_rows=1000)


# speedup vs baseline: 9.8202x; 1.0003x over previous
import functools

import jax
import jax.numpy as jnp
from jax import lax
from jax.experimental import pallas as pl
from jax.experimental.pallas import tpu as pltpu
from jax.experimental.pallas import tpu_sc as plsc

_NC = 2
_NS = 16
_NW = _NC * _NS


def _sc_edge_scatter(table, gidx, sidx, n_rows_out, n_edges, batch):
    d = table.shape[1]
    e_per_w = n_edges // _NW
    n_iter = e_per_w // batch
    n_chunks = 5
    n_sub = n_iter // n_chunks
    rows_per_tile = -(-n_rows_out // (_NS * 8)) * 8
    n_pad = rows_per_tile * _NS

    mesh = plsc.VectorSubcoreMesh(core_axis_name="c", subcore_axis_name="s")

    @functools.partial(
        pl.kernel,
        out_type=jax.ShapeDtypeStruct((_NC, n_pad, d), jnp.float32),
        mesh=mesh,
        scratch_types=[
            pltpu.VMEM((n_sub, batch), jnp.int32),
            pltpu.VMEM((n_sub, batch), jnp.int32),
            pltpu.VMEM((batch, d), jnp.float32),
            pltpu.VMEM((batch, d), jnp.float32),
            pltpu.VMEM_SHARED((n_pad, d), jnp.float32),
            pltpu.SemaphoreType.DMA,
            pltpu.SemaphoreType.DMA,
        ],
    )
    def k(table_hbm, gidx_hbm, sidx_hbm, zeros_hbm, out_hbm,
          gi_v, si_v, rows0, rows1, acc_sh, sem0, sem1):
        c = lax.axis_index("c")
        s = lax.axis_index("s")
        wid = s * _NC + c

        r0 = s * rows_per_tile
        pltpu.sync_copy(zeros_hbm.at[pl.ds(r0, rows_per_tile)],
                        acc_sh.at[pl.ds(r0, rows_per_tile)])
        plsc.subcore_barrier()

        def chunk(u, carry):
            pltpu.sync_copy(gidx_hbm.at[wid, pl.ds(u * n_sub, n_sub)], gi_v)
            pltpu.sync_copy(sidx_hbm.at[wid, pl.ds(u * n_sub, n_sub)], si_v)
            pltpu.async_copy(table_hbm.at[gi_v.at[0]], rows0, sem0)
            pltpu.async_copy(table_hbm.at[gi_v.at[1]], rows1, sem1)

            def body(k2, c2):
                j0 = k2 * 2
                pltpu.make_async_copy(table_hbm.at[gi_v.at[0]], rows0, sem0).wait()
                pltpu.sync_copy(rows0, acc_sh.at[si_v.at[j0]], add=True)
                jn0 = jnp.minimum(j0 + 2, n_sub - 1)
                pltpu.async_copy(table_hbm.at[gi_v.at[jn0]], rows0, sem0)
                pltpu.make_async_copy(table_hbm.at[gi_v.at[0]], rows1, sem1).wait()
                pltpu.sync_copy(rows1, acc_sh.at[si_v.at[j0 + 1]], add=True)
                jn1 = jnp.minimum(j0 + 3, n_sub - 1)
                pltpu.async_copy(table_hbm.at[gi_v.at[jn1]], rows1, sem1)
                return c2

            lax.fori_loop(0, n_sub // 2, body, 0, unroll=2)
            pltpu.make_async_copy(table_hbm.at[gi_v.at[0]], rows0, sem0).wait()
            if n_sub % 2 == 1:
                pltpu.sync_copy(rows0, acc_sh.at[si_v.at[n_sub - 1]], add=True)
            pltpu.make_async_copy(table_hbm.at[gi_v.at[0]], rows1, sem1).wait()
            return carry

        lax.fori_loop(0, n_chunks, chunk, 0)
        plsc.subcore_barrier()

        pltpu.sync_copy(acc_sh.at[pl.ds(r0, rows_per_tile)],
                        out_hbm.at[c, pl.ds(r0, rows_per_tile)])

    return k(table, gidx.reshape(_NW, n_iter, batch), sidx.reshape(_NW, n_iter, batch),
             jnp.zeros((n_pad, d), jnp.float32))


def _tc_linear(x, w, b, block_rows):
    r, d = x.shape

    def body(x_ref, w_ref, b_ref, o_ref):
        o_ref[...] = (
            jnp.dot(x_ref[...], w_ref[...].T, preferred_element_type=jnp.float32)
            + b_ref[...]
        )

    return pl.pallas_call(
        body,
        grid=(r // block_rows,),
        in_specs=[
            pl.BlockSpec((block_rows, d), lambda i: (i, 0)),
            pl.BlockSpec((d, d), lambda i: (0, 0)),
            pl.BlockSpec((1, d), lambda i: (0, 0)),
        ],
        out_specs=pl.BlockSpec((block_rows, d), lambda i: (i, 0)),
        out_shape=jax.ShapeDtypeStruct((r, d), jnp.float32),
    )(x, w, b.reshape(1, d))


def _tc_fuse(acc, x_m, agg, w_l, b_l, w_r, w_out, b_out, bn_weight, bn_bias, block_rows):
    m, d = x_m.shape
    inv = (1.0 + 1e-5) ** -0.5

    def body(acc_ref, xm_ref, ag_ref, wl_ref, bl_ref, wr_ref, wo_ref, bo_ref,
             bnw_ref, bnb_ref, o_ref):
        hs = acc_ref[0] + acc_ref[1]
        xm = xm_ref[...]
        hseq = (
            jnp.dot(ag_ref[...], wl_ref[...].T, preferred_element_type=jnp.float32)
            + bl_ref[...]
            + jnp.dot(xm, wr_ref[...].T, preferred_element_type=jnp.float32)
        )
        wo = wo_ref[...]
        h = (
            jnp.dot(hs, wo[:, :d].T, preferred_element_type=jnp.float32)
            + jnp.dot(xm, wo[:, d:2 * d].T, preferred_element_type=jnp.float32)
            + jnp.dot(hseq, wo[:, 2 * d:].T, preferred_element_type=jnp.float32)
            + bo_ref[...]
        )
        o_ref[...] = h * (bnw_ref[...] * inv) + bnb_ref[...]

    d_out = w_out.shape[0]
    return pl.pallas_call(
        body,
        grid=(m // block_rows,),
        in_specs=[
            pl.BlockSpec((2, block_rows, d), lambda i: (0, i, 0)),
            pl.BlockSpec((block_rows, d), lambda i: (i, 0)),
            pl.BlockSpec((block_rows, d), lambda i: (i, 0)),
            pl.BlockSpec((d, d), lambda i: (0, 0)),
            pl.BlockSpec((1, d), lambda i: (0, 0)),
            pl.BlockSpec((d, d), lambda i: (0, 0)),
            pl.BlockSpec((d_out, 3 * d), lambda i: (0, 0)),
            pl.BlockSpec((1, d_out), lambda i: (0, 0)),
            pl.BlockSpec((1, d_out), lambda i: (0, 0)),
            pl.BlockSpec((1, d_out), lambda i: (0, 0)),
        ],
        out_specs=pl.BlockSpec((block_rows, d_out), lambda i: (i, 0)),
        out_shape=jax.ShapeDtypeStruct((m, d_out), jnp.float32),
    )(acc, x_m, agg, w_l, b_l.reshape(1, d), w_r, w_out, b_out.reshape(1, d_out),
      bn_weight.reshape(1, d_out), bn_bias.reshape(1, d_out))


def _tc_sum_partials(p, r, block_rows):
    d = p.shape[2]

    def body(p_ref, o_ref):
        o_ref[...] = p_ref[0] + p_ref[1]

    return pl.pallas_call(
        body,
        grid=(r // block_rows,),
        in_specs=[pl.BlockSpec((2, block_rows, d), lambda i: (0, i, 0))],
        out_specs=pl.BlockSpec((block_rows, d), lambda i: (i, 0)),
        out_shape=jax.ShapeDtypeStruct((r, d), jnp.float32),
    )(p)


def kernel(x_metrical, x, edge_index, batch, W_neigh, b_neigh, W_l, b_l, W_r,
           W_out, b_out, bn_weight, bn_bias):
    m, d = x_metrical.shape
    n = x.shape[0]
    e = edge_index.shape[1]

    src = edge_index[0].astype(jnp.int32)
    dst = edge_index[1].astype(jnp.int32)

    h_neigh = _tc_linear(x, W_neigh, b_neigh, block_rows=1000)

    acc = _sc_edge_scatter(h_neigh, src, dst, m, e, batch=125)

    agg = jnp.concatenate([jnp.zeros((1, d), jnp.float32), x_metrical[:-1]], axis=0)
    h = _tc_fuse(acc, x_metrical, agg, W_l, b_l, W_r, W_out, b_out,
                 bn_weight, bn_bias, block_rows=1000)

    part = _sc_edge_scatter(h, dst, src, n, e, batch=125)

    return _tc_sum_partials(part, n, block_rows=1000)


# speedup vs baseline: 10.2124x; 1.0399x over previous
import functools

import jax
import jax.numpy as jnp
from jax import lax
from jax.experimental import pallas as pl
from jax.experimental.pallas import tpu as pltpu
from jax.experimental.pallas import tpu_sc as plsc

_NC = 2
_NS = 16
_NW = _NC * _NS


def _sc_edge_scatter(table, ei, gather_row, n_rows_out, n_edges, batch):
    d = table.shape[1]
    scatter_row = 1 - gather_row
    e_per_w = n_edges // _NW
    n_iter = e_per_w // batch
    n_chunks = 5
    n_sub = n_iter // n_chunks
    rows_per_tile = -(-n_rows_out // (_NS * 8)) * 8
    n_pad = rows_per_tile * _NS

    mesh = plsc.VectorSubcoreMesh(core_axis_name="c", subcore_axis_name="s")

    @functools.partial(
        pl.kernel,
        out_type=jax.ShapeDtypeStruct((_NC, n_pad, d), jnp.float32),
        mesh=mesh,
        scratch_types=[
            pltpu.VMEM((n_sub, batch), jnp.int32),
            pltpu.VMEM((n_sub, batch), jnp.int32),
            pltpu.VMEM((batch, d), jnp.float32),
            pltpu.VMEM((batch, d), jnp.float32),
            pltpu.VMEM_SHARED((n_pad, d), jnp.float32),
            pltpu.SemaphoreType.DMA,
            pltpu.SemaphoreType.DMA,
        ],
    )
    def k(table_hbm, ei_hbm, zeros_hbm, out_hbm,
          gi_v, si_v, rows0, rows1, acc_sh, sem0, sem1):
        c = lax.axis_index("c")
        s = lax.axis_index("s")
        wid = s * _NC + c

        r0 = s * rows_per_tile
        pltpu.sync_copy(zeros_hbm.at[pl.ds(r0, rows_per_tile)],
                        acc_sh.at[pl.ds(r0, rows_per_tile)])
        plsc.subcore_barrier()

        def chunk(u, carry):
            pltpu.sync_copy(ei_hbm.at[gather_row, wid, pl.ds(u * n_sub, n_sub)], gi_v)
            pltpu.sync_copy(ei_hbm.at[scatter_row, wid, pl.ds(u * n_sub, n_sub)], si_v)
            pltpu.async_copy(table_hbm.at[gi_v.at[0]], rows0, sem0)
            pltpu.async_copy(table_hbm.at[gi_v.at[1]], rows1, sem1)

            def body(k2, c2):
                j0 = k2 * 2
                pltpu.make_async_copy(table_hbm.at[gi_v.at[0]], rows0, sem0).wait()
                pltpu.sync_copy(rows0, acc_sh.at[si_v.at[j0]], add=True)
                jn0 = jnp.minimum(j0 + 2, n_sub - 1)
                pltpu.async_copy(table_hbm.at[gi_v.at[jn0]], rows0, sem0)
                pltpu.make_async_copy(table_hbm.at[gi_v.at[0]], rows1, sem1).wait()
                pltpu.sync_copy(rows1, acc_sh.at[si_v.at[j0 + 1]], add=True)
                jn1 = jnp.minimum(j0 + 3, n_sub - 1)
                pltpu.async_copy(table_hbm.at[gi_v.at[jn1]], rows1, sem1)
                return c2

            lax.fori_loop(0, n_sub // 2, body, 0, unroll=2)
            pltpu.make_async_copy(table_hbm.at[gi_v.at[0]], rows0, sem0).wait()
            if n_sub % 2 == 1:
                pltpu.sync_copy(rows0, acc_sh.at[si_v.at[n_sub - 1]], add=True)
            pltpu.make_async_copy(table_hbm.at[gi_v.at[0]], rows1, sem1).wait()
            return carry

        lax.fori_loop(0, n_chunks, chunk, 0)
        plsc.subcore_barrier()

        pltpu.sync_copy(acc_sh.at[pl.ds(r0, rows_per_tile)],
                        out_hbm.at[c, pl.ds(r0, rows_per_tile)])

    return k(table, ei.reshape(2, _NW, n_iter, batch), jnp.zeros((n_pad, d), jnp.float32))


def _tc_linear(x, w, b, block_rows):
    r, d = x.shape

    def body(x_ref, w_ref, b_ref, o_ref):
        o_ref[...] = (
            jnp.dot(x_ref[...], w_ref[...].T, preferred_element_type=jnp.float32)
            + b_ref[...]
        )

    return pl.pallas_call(
        body,
        grid=(r // block_rows,),
        in_specs=[
            pl.BlockSpec((block_rows, d), lambda i: (i, 0)),
            pl.BlockSpec((d, d), lambda i: (0, 0)),
            pl.BlockSpec((1, d), lambda i: (0, 0)),
        ],
        out_specs=pl.BlockSpec((block_rows, d), lambda i: (i, 0)),
        out_shape=jax.ShapeDtypeStruct((r, d), jnp.float32),
    )(x, w, b.reshape(1, d))


def _tc_fuse(acc, x_m, w_l, b_l, w_r, w_out, b_out, bn_weight, bn_bias, block_rows):
    m, d = x_m.shape
    inv = (1.0 + 1e-5) ** -0.5

    def body(acc_ref, xm_ref, bd_ref, wl_ref, bl_ref, wr_ref, wo_ref, bo_ref,
             bnw_ref, bnb_ref, o_ref):
        i = pl.program_id(0)
        hs = acc_ref[0] + acc_ref[1]
        xm = xm_ref[...]
        first = jnp.where(i == 0, 0.0, bd_ref[7:8, :])
        ag = jnp.concatenate([first, xm[:-1, :]], axis=0)
        hseq = (
            jnp.dot(ag, wl_ref[...].T, preferred_element_type=jnp.float32)
            + bl_ref[...]
            + jnp.dot(xm, wr_ref[...].T, preferred_element_type=jnp.float32)
        )
        wo = wo_ref[...]
        h = (
            jnp.dot(hs, wo[:, :d].T, preferred_element_type=jnp.float32)
            + jnp.dot(xm, wo[:, d:2 * d].T, preferred_element_type=jnp.float32)
            + jnp.dot(hseq, wo[:, 2 * d:].T, preferred_element_type=jnp.float32)
            + bo_ref[...]
        )
        o_ref[...] = h * (bnw_ref[...] * inv) + bnb_ref[...]

    d_out = w_out.shape[0]
    return pl.pallas_call(
        body,
        grid=(m // block_rows,),
        in_specs=[
            pl.BlockSpec((2, block_rows, d), lambda i: (0, i, 0)),
            pl.BlockSpec((block_rows, d), lambda i: (i, 0)),
            pl.BlockSpec((8, d), lambda i: (jnp.maximum(i * (block_rows // 8) - 1, 0), 0)),
            pl.BlockSpec((d, d), lambda i: (0, 0)),
            pl.BlockSpec((1, d), lambda i: (0, 0)),
            pl.BlockSpec((d, d), lambda i: (0, 0)),
            pl.BlockSpec((d_out, 3 * d), lambda i: (0, 0)),
            pl.BlockSpec((1, d_out), lambda i: (0, 0)),
            pl.BlockSpec((1, d_out), lambda i: (0, 0)),
            pl.BlockSpec((1, d_out), lambda i: (0, 0)),
        ],
        out_specs=pl.BlockSpec((block_rows, d_out), lambda i: (i, 0)),
        out_shape=jax.ShapeDtypeStruct((m, d_out), jnp.float32),
    )(acc, x_m, x_m, w_l, b_l.reshape(1, d), w_r, w_out, b_out.reshape(1, d_out),
      bn_weight.reshape(1, d_out), bn_bias.reshape(1, d_out))


def _tc_sum_partials(p, r, block_rows):
    d = p.shape[2]

    def body(p_ref, o_ref):
        o_ref[...] = p_ref[0] + p_ref[1]

    return pl.pallas_call(
        body,
        grid=(r // block_rows,),
        in_specs=[pl.BlockSpec((2, block_rows, d), lambda i: (0, i, 0))],
        out_specs=pl.BlockSpec((block_rows, d), lambda i: (i, 0)),
        out_shape=jax.ShapeDtypeStruct((r, d), jnp.float32),
    )(p)


def kernel(x_metrical, x, edge_index, batch, W_neigh, b_neigh, W_l, b_l, W_r,
           W_out, b_out, bn_weight, bn_bias):
    m, d = x_metrical.shape
    n = x.shape[0]
    e = edge_index.shape[1]

    ei = edge_index.astype(jnp.int32)

    h_neigh = _tc_linear(x, W_neigh, b_neigh, block_rows=1000)

    acc = _sc_edge_scatter(h_neigh, ei, 0, m, e, batch=125)

    h = _tc_fuse(acc, x_metrical, W_l, b_l, W_r, W_out, b_out,
                 bn_weight, bn_bias, block_rows=1000)

    part = _sc_edge_scatter(h, ei, 1, n, e, batch=125)

    return _tc_sum_partials(part, n, block_rows=1000)


# speedup vs baseline: 11.4539x; 1.1216x over previous
import functools

import jax
import jax.numpy as jnp
from jax import lax
from jax.experimental import pallas as pl
from jax.experimental.pallas import tpu as pltpu
from jax.experimental.pallas import tpu_sc as plsc

_NC = 2
_NS = 16
_NW = _NC * _NS


def _sc_edge_scatter(table, ei, gather_row, n_rows_out, n_edges, batch):
    d = table.shape[1]
    scatter_row = 1 - gather_row
    e_per_w = n_edges // _NW
    n_iter = e_per_w // batch
    n_chunks = 5
    n_sub = n_iter // n_chunks
    rows_per_tile = -(-n_rows_out // (_NS * 8)) * 8
    n_pad = rows_per_tile * _NS

    mesh = plsc.VectorSubcoreMesh(core_axis_name="c", subcore_axis_name="s")

    nbuf = 3
    n_trip = n_sub // nbuf
    scratch = [
        pltpu.VMEM((n_sub, batch), jnp.int32),
        pltpu.VMEM((n_sub, batch), jnp.int32),
    ] + [pltpu.VMEM((batch, d), jnp.float32) for _ in range(nbuf)] + [
        pltpu.VMEM_SHARED((n_pad, d), jnp.float32),
    ] + [pltpu.SemaphoreType.DMA for _ in range(nbuf)]

    @functools.partial(
        pl.kernel,
        out_type=jax.ShapeDtypeStruct((_NC, n_pad, d), jnp.float32),
        mesh=mesh,
        scratch_types=scratch,
    )
    def k(table_hbm, ei_hbm, zeros_hbm, out_hbm,
          gi_v, si_v, rows0, rows1, rows2, acc_sh, sem0, sem1, sem2):
        rows = (rows0, rows1, rows2)
        sems = (sem0, sem1, sem2)
        c = lax.axis_index("c")
        s = lax.axis_index("s")
        wid = s * _NC + c

        r0 = s * rows_per_tile
        pltpu.sync_copy(zeros_hbm.at[pl.ds(r0, rows_per_tile)],
                        acc_sh.at[pl.ds(r0, rows_per_tile)])
        plsc.subcore_barrier()

        def chunk(u, carry):
            pltpu.sync_copy(ei_hbm.at[gather_row, wid, u], gi_v)
            pltpu.sync_copy(ei_hbm.at[scatter_row, wid, u], si_v)
            for b in range(nbuf):
                pltpu.async_copy(table_hbm.at[gi_v.at[b]], rows[b], sems[b])

            def body(t, c2):
                j0 = t * nbuf
                for b in range(nbuf):
                    j = j0 + b
                    pltpu.make_async_copy(table_hbm.at[gi_v.at[0]], rows[b], sems[b]).wait()
                    pltpu.sync_copy(rows[b], acc_sh.at[si_v.at[j]], add=True)
                    jn = j + nbuf

                    @pl.when(jn < n_sub)
                    def _():
                        pltpu.async_copy(table_hbm.at[gi_v.at[jn]], rows[b], sems[b])
                return c2

            lax.fori_loop(0, n_trip, body, 0)
            for b in range(nbuf):
                j = n_trip * nbuf + b
                if j < n_sub:
                    pltpu.make_async_copy(table_hbm.at[gi_v.at[0]], rows[b], sems[b]).wait()
                    pltpu.sync_copy(rows[b], acc_sh.at[si_v.at[j]], add=True)
            return carry

        lax.fori_loop(0, n_chunks, chunk, 0)
        plsc.subcore_barrier()

        pltpu.sync_copy(acc_sh.at[pl.ds(r0, rows_per_tile)],
                        out_hbm.at[c, pl.ds(r0, rows_per_tile)])

    return k(table, ei.reshape(2, _NW, n_chunks, n_sub, batch),
             jnp.zeros((n_pad, d), jnp.float32))


def _tc_linear(x, w, b, block_rows):
    r, d = x.shape

    def body(x_ref, w_ref, b_ref, o_ref):
        o_ref[...] = (
            jnp.dot(x_ref[...], w_ref[...].T, preferred_element_type=jnp.float32)
            + b_ref[...]
        )

    return pl.pallas_call(
        body,
        grid=(r // block_rows,),
        in_specs=[
            pl.BlockSpec((block_rows, d), lambda i: (i, 0)),
            pl.BlockSpec((d, d), lambda i: (0, 0)),
            pl.BlockSpec((1, d), lambda i: (0, 0)),
        ],
        out_specs=pl.BlockSpec((block_rows, d), lambda i: (i, 0)),
        out_shape=jax.ShapeDtypeStruct((r, d), jnp.float32),
    )(x, w, b.reshape(1, d))


def _tc_fuse(acc, x_m, w_l, b_l, w_r, w_out, b_out, bn_weight, bn_bias, block_rows):
    m, d = x_m.shape
    inv = (1.0 + 1e-5) ** -0.5

    def body(acc_ref, xm_ref, bd_ref, wl_ref, bl_ref, wr_ref, wo_ref, bo_ref,
             bnw_ref, bnb_ref, o_ref):
        i = pl.program_id(0)
        hs = acc_ref[0] + acc_ref[1]
        xm = xm_ref[...]
        first = jnp.where(i == 0, 0.0, bd_ref[7:8, :])
        ag = jnp.concatenate([first, xm[:-1, :]], axis=0)
        hseq = (
            jnp.dot(ag, wl_ref[...].T, preferred_element_type=jnp.float32)
            + bl_ref[...]
            + jnp.dot(xm, wr_ref[...].T, preferred_element_type=jnp.float32)
        )
        wo = wo_ref[...]
        h = (
            jnp.dot(hs, wo[:, :d].T, preferred_element_type=jnp.float32)
            + jnp.dot(xm, wo[:, d:2 * d].T, preferred_element_type=jnp.float32)
            + jnp.dot(hseq, wo[:, 2 * d:].T, preferred_element_type=jnp.float32)
            + bo_ref[...]
        )
        o_ref[...] = h * (bnw_ref[...] * inv) + bnb_ref[...]

    d_out = w_out.shape[0]
    return pl.pallas_call(
        body,
        grid=(m // block_rows,),
        in_specs=[
            pl.BlockSpec((2, block_rows, d), lambda i: (0, i, 0)),
            pl.BlockSpec((block_rows, d), lambda i: (i, 0)),
            pl.BlockSpec((8, d), lambda i: (jnp.maximum(i * (block_rows // 8) - 1, 0), 0)),
            pl.BlockSpec((d, d), lambda i: (0, 0)),
            pl.BlockSpec((1, d), lambda i: (0, 0)),
            pl.BlockSpec((d, d), lambda i: (0, 0)),
            pl.BlockSpec((d_out, 3 * d), lambda i: (0, 0)),
            pl.BlockSpec((1, d_out), lambda i: (0, 0)),
            pl.BlockSpec((1, d_out), lambda i: (0, 0)),
            pl.BlockSpec((1, d_out), lambda i: (0, 0)),
        ],
        out_specs=pl.BlockSpec((block_rows, d_out), lambda i: (i, 0)),
        out_shape=jax.ShapeDtypeStruct((m, d_out), jnp.float32),
    )(acc, x_m, x_m, w_l, b_l.reshape(1, d), w_r, w_out, b_out.reshape(1, d_out),
      bn_weight.reshape(1, d_out), bn_bias.reshape(1, d_out))


def _tc_sum_partials(p, r, block_rows):
    d = p.shape[2]

    def body(p_ref, o_ref):
        o_ref[...] = p_ref[0] + p_ref[1]

    return pl.pallas_call(
        body,
        grid=(r // block_rows,),
        in_specs=[pl.BlockSpec((2, block_rows, d), lambda i: (0, i, 0))],
        out_specs=pl.BlockSpec((block_rows, d), lambda i: (i, 0)),
        out_shape=jax.ShapeDtypeStruct((r, d), jnp.float32),
    )(p)


def kernel(x_metrical, x, edge_index, batch, W_neigh, b_neigh, W_l, b_l, W_r,
           W_out, b_out, bn_weight, bn_bias):
    m, d = x_metrical.shape
    n = x.shape[0]
    e = edge_index.shape[1]

    ei = edge_index.astype(jnp.int32)

    h_neigh = _tc_linear(x, W_neigh, b_neigh, block_rows=1000)

    acc = _sc_edge_scatter(h_neigh, ei, 0, m, e, batch=80)

    h = _tc_fuse(acc, x_metrical, W_l, b_l, W_r, W_out, b_out,
                 bn_weight, bn_bias, block_rows=1000)

    part = _sc_edge_scatter(h, ei, 1, n, e, batch=80)

    return _tc_sum_partials(part, n, block_rows=1000)


# speedup vs baseline: 11.5640x; 1.0096x over previous
import functools

import jax
import jax.numpy as jnp
from jax import lax
from jax.experimental import pallas as pl
from jax.experimental.pallas import tpu as pltpu
from jax.experimental.pallas import tpu_sc as plsc

_NC = 2
_NS = 16
_NW = _NC * _NS


def _sc_edge_scatter(table, ei, gather_row, n_rows_out, n_edges, batch):
    d = table.shape[1]
    scatter_row = 1 - gather_row
    e_per_w = n_edges // _NW
    n_iter = e_per_w // batch
    n_chunks = 5
    n_sub = n_iter // n_chunks
    rows_per_tile = -(-n_rows_out // (_NS * 8)) * 8
    n_pad = rows_per_tile * _NS

    mesh = plsc.VectorSubcoreMesh(core_axis_name="c", subcore_axis_name="s")

    nbuf = 4
    n_trip = n_sub // nbuf
    scratch = [
        pltpu.VMEM((n_sub, batch), jnp.int32),
        pltpu.VMEM((n_sub, batch), jnp.int32),
    ] + [pltpu.VMEM((batch, d), jnp.float32) for _ in range(nbuf)] + [
        pltpu.VMEM_SHARED((n_pad, d), jnp.float32),
    ] + [pltpu.SemaphoreType.DMA for _ in range(nbuf)]

    @functools.partial(
        pl.kernel,
        out_type=jax.ShapeDtypeStruct((_NC, n_pad, d), jnp.float32),
        mesh=mesh,
        scratch_types=scratch,
    )
    def k(table_hbm, ei_hbm, zeros_hbm, out_hbm,
          gi_v, si_v, rows0, rows1, rows2, rows3, acc_sh, sem0, sem1, sem2, sem3):
        rows = (rows0, rows1, rows2, rows3)
        sems = (sem0, sem1, sem2, sem3)
        c = lax.axis_index("c")
        s = lax.axis_index("s")
        wid = s * _NC + c

        r0 = s * rows_per_tile
        pltpu.sync_copy(zeros_hbm.at[pl.ds(r0, rows_per_tile)],
                        acc_sh.at[pl.ds(r0, rows_per_tile)])
        plsc.subcore_barrier()

        def chunk(u, carry):
            pltpu.sync_copy(ei_hbm.at[gather_row, wid, u], gi_v)
            pltpu.sync_copy(ei_hbm.at[scatter_row, wid, u], si_v)
            for b in range(nbuf):
                pltpu.async_copy(table_hbm.at[gi_v.at[b]], rows[b], sems[b])

            def body(t, c2):
                j0 = t * nbuf
                for b in range(nbuf):
                    j = j0 + b
                    pltpu.make_async_copy(table_hbm.at[gi_v.at[0]], rows[b], sems[b]).wait()
                    pltpu.sync_copy(rows[b], acc_sh.at[si_v.at[j]], add=True)
                    jn = j + nbuf

                    @pl.when(jn < n_sub)
                    def _():
                        pltpu.async_copy(table_hbm.at[gi_v.at[jn]], rows[b], sems[b])
                return c2

            lax.fori_loop(0, n_trip, body, 0)
            for b in range(nbuf):
                j = n_trip * nbuf + b
                if j < n_sub:
                    pltpu.make_async_copy(table_hbm.at[gi_v.at[0]], rows[b], sems[b]).wait()
                    pltpu.sync_copy(rows[b], acc_sh.at[si_v.at[j]], add=True)
            return carry

        lax.fori_loop(0, n_chunks, chunk, 0)
        plsc.subcore_barrier()

        pltpu.sync_copy(acc_sh.at[pl.ds(r0, rows_per_tile)],
                        out_hbm.at[c, pl.ds(r0, rows_per_tile)])

    return k(table, ei.reshape(2, _NW, n_chunks, n_sub, batch),
             jnp.zeros((n_pad, d), jnp.float32))


def _tc_linear(x, w, b, block_rows):
    r, d = x.shape

    def body(x_ref, w_ref, b_ref, o_ref):
        o_ref[...] = (
            jnp.dot(x_ref[...], w_ref[...].T, preferred_element_type=jnp.float32)
            + b_ref[...]
        )

    return pl.pallas_call(
        body,
        grid=(r // block_rows,),
        in_specs=[
            pl.BlockSpec((block_rows, d), lambda i: (i, 0)),
            pl.BlockSpec((d, d), lambda i: (0, 0)),
            pl.BlockSpec((1, d), lambda i: (0, 0)),
        ],
        out_specs=pl.BlockSpec((block_rows, d), lambda i: (i, 0)),
        out_shape=jax.ShapeDtypeStruct((r, d), jnp.float32),
    )(x, w, b.reshape(1, d))


def _tc_fuse(acc, x_m, w_l, b_l, w_r, w_out, b_out, bn_weight, bn_bias, block_rows):
    m, d = x_m.shape
    inv = (1.0 + 1e-5) ** -0.5

    def body(acc_ref, xm_ref, bd_ref, wl_ref, bl_ref, wr_ref, wo_ref, bo_ref,
             bnw_ref, bnb_ref, o_ref):
        i = pl.program_id(0)
        hs = acc_ref[0] + acc_ref[1]
        xm = xm_ref[...]
        first = jnp.where(i == 0, 0.0, bd_ref[7:8, :])
        ag = jnp.concatenate([first, xm[:-1, :]], axis=0)
        hseq = (
            jnp.dot(ag, wl_ref[...].T, preferred_element_type=jnp.float32)
            + bl_ref[...]
            + jnp.dot(xm, wr_ref[...].T, preferred_element_type=jnp.float32)
        )
        wo = wo_ref[...]
        h = (
            jnp.dot(hs, wo[:, :d].T, preferred_element_type=jnp.float32)
            + jnp.dot(xm, wo[:, d:2 * d].T, preferred_element_type=jnp.float32)
            + jnp.dot(hseq, wo[:, 2 * d:].T, preferred_element_type=jnp.float32)
            + bo_ref[...]
        )
        o_ref[...] = h * (bnw_ref[...] * inv) + bnb_ref[...]

    d_out = w_out.shape[0]
    return pl.pallas_call(
        body,
        grid=(m // block_rows,),
        in_specs=[
            pl.BlockSpec((2, block_rows, d), lambda i: (0, i, 0)),
            pl.BlockSpec((block_rows, d), lambda i: (i, 0)),
            pl.BlockSpec((8, d), lambda i: (jnp.maximum(i * (block_rows // 8) - 1, 0), 0)),
            pl.BlockSpec((d, d), lambda i: (0, 0)),
            pl.BlockSpec((1, d), lambda i: (0, 0)),
            pl.BlockSpec((d, d), lambda i: (0, 0)),
            pl.BlockSpec((d_out, 3 * d), lambda i: (0, 0)),
            pl.BlockSpec((1, d_out), lambda i: (0, 0)),
            pl.BlockSpec((1, d_out), lambda i: (0, 0)),
            pl.BlockSpec((1, d_out), lambda i: (0, 0)),
        ],
        out_specs=pl.BlockSpec((block_rows, d_out), lambda i: (i, 0)),
        out_shape=jax.ShapeDtypeStruct((m, d_out), jnp.float32),
    )(acc, x_m, x_m, w_l, b_l.reshape(1, d), w_r, w_out, b_out.reshape(1, d_out),
      bn_weight.reshape(1, d_out), bn_bias.reshape(1, d_out))


def _tc_sum_partials(p, r, block_rows):
    d = p.shape[2]

    def body(p_ref, o_ref):
        o_ref[...] = p_ref[0] + p_ref[1]

    return pl.pallas_call(
        body,
        grid=(r // block_rows,),
        in_specs=[pl.BlockSpec((2, block_rows, d), lambda i: (0, i, 0))],
        out_specs=pl.BlockSpec((block_rows, d), lambda i: (i, 0)),
        out_shape=jax.ShapeDtypeStruct((r, d), jnp.float32),
    )(p)


def kernel(x_metrical, x, edge_index, batch, W_neigh, b_neigh, W_l, b_l, W_r,
           W_out, b_out, bn_weight, bn_bias):
    m, d = x_metrical.shape
    n = x.shape[0]
    e = edge_index.shape[1]

    ei = edge_index.astype(jnp.int32)

    h_neigh = _tc_linear(x, W_neigh, b_neigh, block_rows=1000)

    acc = _sc_edge_scatter(h_neigh, ei, 0, m, e, batch=80)

    h = _tc_fuse(acc, x_metrical, W_l, b_l, W_r, W_out, b_out,
                 bn_weight, bn_bias, block_rows=1000)

    part = _sc_edge_scatter(h, ei, 1, n, e, batch=80)

    return _tc_sum_partials(part, n, block_rows=1000)


# speedup vs baseline: 11.6202x; 1.0049x over previous
import functools

import jax
import jax.numpy as jnp
from jax import lax
from jax.experimental import pallas as pl
from jax.experimental.pallas import tpu as pltpu
from jax.experimental.pallas import tpu_sc as plsc

_NC = 2
_NS = 16
_NW = _NC * _NS


def _sc_edge_scatter(table, ei, gather_row, n_rows_out, n_edges, batch):
    d = table.shape[1]
    scatter_row = 1 - gather_row
    e_per_w = n_edges // _NW
    n_iter = e_per_w // batch
    n_chunks = 5
    n_sub = n_iter // n_chunks
    rows_per_tile = -(-n_rows_out // (_NS * 8)) * 8
    n_pad = rows_per_tile * _NS

    mesh = plsc.VectorSubcoreMesh(core_axis_name="c", subcore_axis_name="s")

    nbuf = 4
    n_trip = n_sub // nbuf
    scratch = [
        pltpu.VMEM((n_sub, batch), jnp.int32),
        pltpu.VMEM((n_sub, batch), jnp.int32),
    ] + [pltpu.VMEM((batch, d), jnp.float32) for _ in range(nbuf)] + [
        pltpu.VMEM_SHARED((n_pad, d), jnp.float32),
    ] + [pltpu.SemaphoreType.DMA for _ in range(nbuf)]

    @functools.partial(
        pl.kernel,
        out_type=jax.ShapeDtypeStruct((_NC, n_pad, d), jnp.float32),
        mesh=mesh,
        scratch_types=scratch,
    )
    def k(table_hbm, ei_hbm, zeros_hbm, out_hbm,
          gi_v, si_v, rows0, rows1, rows2, rows3, acc_sh, sem0, sem1, sem2, sem3):
        rows = (rows0, rows1, rows2, rows3)
        sems = (sem0, sem1, sem2, sem3)
        c = lax.axis_index("c")
        s = lax.axis_index("s")
        wid = s * _NC + c

        r0 = s * rows_per_tile
        pltpu.sync_copy(zeros_hbm.at[pl.ds(r0, rows_per_tile)],
                        acc_sh.at[pl.ds(r0, rows_per_tile)])
        plsc.subcore_barrier()

        def chunk(u, carry):
            pltpu.sync_copy(ei_hbm.at[gather_row, wid, u], gi_v)
            pltpu.sync_copy(ei_hbm.at[scatter_row, wid, u], si_v)
            for b in range(nbuf):
                pltpu.async_copy(table_hbm.at[gi_v.at[b]], rows[b], sems[b])

            def body(t, c2):
                j0 = t * nbuf
                for b in range(nbuf):
                    j = j0 + b
                    pltpu.make_async_copy(table_hbm.at[gi_v.at[0]], rows[b], sems[b]).wait()
                    pltpu.sync_copy(rows[b], acc_sh.at[si_v.at[j]], add=True)
                    jn = j + nbuf

                    @pl.when(jn < n_sub)
                    def _():
                        pltpu.async_copy(table_hbm.at[gi_v.at[jn]], rows[b], sems[b])
                return c2

            lax.fori_loop(0, n_trip, body, 0)
            for b in range(nbuf):
                j = n_trip * nbuf + b
                if j < n_sub:
                    pltpu.make_async_copy(table_hbm.at[gi_v.at[0]], rows[b], sems[b]).wait()
                    pltpu.sync_copy(rows[b], acc_sh.at[si_v.at[j]], add=True)
            return carry

        lax.fori_loop(0, n_chunks, chunk, 0)
        plsc.subcore_barrier()

        pltpu.sync_copy(acc_sh.at[pl.ds(r0, rows_per_tile)],
                        out_hbm.at[c, pl.ds(r0, rows_per_tile)])

    return k(table, ei.reshape(2, _NW, n_chunks, n_sub, batch),
             jnp.zeros((n_pad, d), jnp.float32))


def _tc_linear(x, w, b, block_rows):
    r, d = x.shape

    def body(x_ref, w_ref, b_ref, o_ref):
        o_ref[...] = (
            jnp.dot(x_ref[...], w_ref[...].T, preferred_element_type=jnp.float32)
            + b_ref[...]
        )

    return pl.pallas_call(
        body,
        grid=(r // block_rows,),
        in_specs=[
            pl.BlockSpec((block_rows, d), lambda i: (i, 0)),
            pl.BlockSpec((d, d), lambda i: (0, 0)),
            pl.BlockSpec((1, d), lambda i: (0, 0)),
        ],
        out_specs=pl.BlockSpec((block_rows, d), lambda i: (i, 0)),
        out_shape=jax.ShapeDtypeStruct((r, d), jnp.float32),
    )(x, w, b.reshape(1, d))


def _tc_fuse(acc, x_m, a1, a2, a3, bias, block_rows):
    m, d = x_m.shape
    d_out = a1.shape[1]

    def body(acc_ref, xm_ref, bd_ref, a1_ref, a2_ref, a3_ref, b_ref, o_ref):
        i = pl.program_id(0)
        hs = acc_ref[0] + acc_ref[1]
        xm = xm_ref[...]
        first = jnp.where(i == 0, 0.0, bd_ref[7:8, :])
        ag = jnp.concatenate([first, xm[:-1, :]], axis=0)
        o_ref[...] = (
            jnp.dot(hs, a1_ref[...], preferred_element_type=jnp.float32)
            + jnp.dot(xm, a2_ref[...], preferred_element_type=jnp.float32)
            + jnp.dot(ag, a3_ref[...], preferred_element_type=jnp.float32)
            + b_ref[...]
        )

    return pl.pallas_call(
        body,
        grid=(m // block_rows,),
        in_specs=[
            pl.BlockSpec((2, block_rows, d), lambda i: (0, i, 0)),
            pl.BlockSpec((block_rows, d), lambda i: (i, 0)),
            pl.BlockSpec((8, d), lambda i: (jnp.maximum(i * (block_rows // 8) - 1, 0), 0)),
            pl.BlockSpec((d, d_out), lambda i: (0, 0)),
            pl.BlockSpec((d, d_out), lambda i: (0, 0)),
            pl.BlockSpec((d, d_out), lambda i: (0, 0)),
            pl.BlockSpec((1, d_out), lambda i: (0, 0)),
        ],
        out_specs=pl.BlockSpec((block_rows, d_out), lambda i: (i, 0)),
        out_shape=jax.ShapeDtypeStruct((m, d_out), jnp.float32),
    )(acc, x_m, x_m, a1, a2, a3, bias.reshape(1, d_out))


def _tc_sum_partials(p, r, block_rows):
    d = p.shape[2]

    def body(p_ref, o_ref):
        o_ref[...] = p_ref[0] + p_ref[1]

    return pl.pallas_call(
        body,
        grid=(r // block_rows,),
        in_specs=[pl.BlockSpec((2, block_rows, d), lambda i: (0, i, 0))],
        out_specs=pl.BlockSpec((block_rows, d), lambda i: (i, 0)),
        out_shape=jax.ShapeDtypeStruct((r, d), jnp.float32),
    )(p)


def kernel(x_metrical, x, edge_index, batch, W_neigh, b_neigh, W_l, b_l, W_r,
           W_out, b_out, bn_weight, bn_bias):
    m, d = x_metrical.shape
    n = x.shape[0]
    e = edge_index.shape[1]

    ei = edge_index.astype(jnp.int32)

    h_neigh = _tc_linear(x, W_neigh, b_neigh, block_rows=1000)

    acc = _sc_edge_scatter(h_neigh, ei, 0, m, e, batch=80)

    inv = (1.0 + 1e-5) ** -0.5
    sc = (bn_weight * inv)[None, :]
    wo3t = W_out[:, 2 * d:].T
    a1 = W_out[:, :d].T * sc
    a2 = (W_out[:, d:2 * d].T + W_r.T @ wo3t) * sc
    a3 = (W_l.T @ wo3t) * sc
    bias = (b_l @ wo3t + b_out) * sc[0] + bn_bias
    h = _tc_fuse(acc, x_metrical, a1, a2, a3, bias, block_rows=1000)

    part = _sc_edge_scatter(h, ei, 1, n, e, batch=80)

    return _tc_sum_partials(part, n, block_rows=1000)
